# baseline (speedup 1.0000x reference)
import jax
import jax.numpy as jnp
from jax.experimental import pallas as pl
from jax.experimental.pallas import tpu as pltpu

_U, _I = 10000, 5000
_BU = 80
_NU = _U // _BU


def _probe_kernel(a_ref, o_ref):
    o_ref[...] = jnp.sum(a_ref[...], axis=1, keepdims=True) + jnp.zeros((_BU, 128), jnp.float32)


def kernel(adj, recovery_stage_idx, preferred_type_idx, resource_type_idx,
           user_emb_w, item_emb_w, recovery_emb_w, type_emb_w,
           resource_type_emb_w, user_proj_w, user_proj_b, item_proj_w,
           item_proj_b):
    o = pl.pallas_call(
        _probe_kernel,
        grid=(3, _NU),
        in_specs=[pl.BlockSpec((_BU, _I), lambda l, u: (u, 0))],
        out_specs=pl.BlockSpec((_BU, 128), lambda l, u: (u, 0)),
        out_shape=jax.ShapeDtypeStruct((_U, 128), jnp.float32),
        compiler_params=pltpu.CompilerParams(
            dimension_semantics=("arbitrary", "arbitrary")),
    )(adj)
    z = o[:, :32]
    return (z, z[:5000])


# speedup vs baseline: 1.1688x; 1.1688x over previous
import jax
import jax.numpy as jnp
from jax.experimental import pallas as pl
from jax.experimental.pallas import tpu as pltpu

_U, _I = 10000, 5000
_D, _F = 32, 8
_REC_V, _TYP_V, _RES_V = 8, 8, 16
_L = 3
_BU = 400
_NU = _U // _BU


def _mm_t(x, w):
    return jax.lax.dot_general(x, w, (((1,), (1,)), ((), ())),
                               preferred_element_type=jnp.float32)


def _mm_ct(x, w):
    return jax.lax.dot_general(x, w, (((0,), (0,)), ((), ())),
                               preferred_element_type=jnp.float32)


def _gcn_kernel(adj_ref, rec_idx_ref, typ_idx_ref, res_idx_ref,
                ue_ref, ie_ref, rec_w_ref, typ_w_ref, res_w_ref,
                wu_ref, bu_ref, wi_ref, bi_ref,
                uo_ref, io_ref,
                cur_u, cur_it, nxt_u, nxt_it):
    l = pl.program_id(0)
    ub = pl.program_id(1)

    @pl.when(ub == 0)
    def _layer_start():
        @pl.when(l == 0)
        def _enrich():
            wu = wu_ref[...]
            t_rec = _mm_t(rec_w_ref[...], wu[:, _D:_D + _F])
            t_typ = _mm_t(typ_w_ref[...], wu[:, _D + _F:])
            oh_rec = (rec_idx_ref[...] == jax.lax.broadcasted_iota(
                jnp.int32, (_REC_V, _U), 0)).astype(jnp.float32)
            oh_typ = (typ_idx_ref[...] == jax.lax.broadcasted_iota(
                jnp.int32, (_TYP_V, _U), 0)).astype(jnp.float32)
            eu = (_mm_t(ue_ref[...], wu[:, :_D])
                  + _mm_ct(oh_rec, t_rec) + _mm_ct(oh_typ, t_typ)
                  + bu_ref[...])
            wi = wi_ref[...]
            t_res = _mm_t(res_w_ref[...], wi[:, _D:])
            oh_res = (res_idx_ref[...] == jax.lax.broadcasted_iota(
                jnp.int32, (_RES_V, _I), 0)).astype(jnp.float32)
            ei = (_mm_t(ie_ref[...], wi[:, :_D])
                  + _mm_ct(oh_res, t_res) + bi_ref[...])
            cur_u[...] = eu
            cur_it[...] = ei
            uo_ref[...] = eu
            io_ref[...] = ei

        @pl.when(l > 0)
        def _advance():
            u = nxt_u[...]
            it = nxt_it[...]
            cur_u[...] = u
            cur_it[...] = it
            uo_ref[...] += u
            io_ref[...] += it

        nxt_u[...] = jnp.zeros_like(nxt_u)
        nxt_it[...] = jnp.zeros_like(nxt_it)

    a = adj_ref[...].astype(jnp.bfloat16)
    u_blk = cur_u[pl.ds(ub * _BU, _BU), :].astype(jnp.bfloat16)
    nxt_u[pl.ds(ub * _BU, _BU), :] = jnp.dot(
        a, cur_it[...].astype(jnp.bfloat16), preferred_element_type=jnp.float32)
    nxt_it[...] += _mm_ct(a, u_blk)

    @pl.when((l == _L - 1) & (ub == _NU - 1))
    def _finish():
        uo_ref[...] = (uo_ref[...] + nxt_u[...]) * (1.0 / (_L + 1))
        io_ref[...] = (io_ref[...] + nxt_it[...]) * (1.0 / (_L + 1))


def _full(shape):
    return pl.BlockSpec(shape, lambda l, u: (0,) * len(shape))


def kernel(adj, recovery_stage_idx, preferred_type_idx, resource_type_idx,
           user_emb_w, item_emb_w, recovery_emb_w, type_emb_w,
           resource_type_emb_w, user_proj_w, user_proj_b, item_proj_w,
           item_proj_b):
    rec2 = recovery_stage_idx.astype(jnp.int32).reshape(1, _U)
    typ2 = preferred_type_idx.astype(jnp.int32).reshape(1, _U)
    res2 = resource_type_idx.astype(jnp.int32).reshape(1, _I)
    bu2 = user_proj_b.reshape(1, _D)
    bi2 = item_proj_b.reshape(1, _D)

    user_out, item_out = pl.pallas_call(
        _gcn_kernel,
        grid=(_L, _NU),
        in_specs=[
            pl.BlockSpec((_BU, _I), lambda l, u: (u, 0)),
            _full((1, _U)), _full((1, _U)), _full((1, _I)),
            _full((_U, _D)), _full((_I, _D)),
            _full((_REC_V, _F)), _full((_TYP_V, _F)), _full((_RES_V, _F)),
            _full((_D, _D + 2 * _F)), _full((1, _D)),
            _full((_D, _D + _F)), _full((1, _D)),
        ],
        out_specs=[_full((_U, _D)), _full((_I, _D))],
        out_shape=[jax.ShapeDtypeStruct((_U, _D), jnp.float32),
                   jax.ShapeDtypeStruct((_I, _D), jnp.float32)],
        scratch_shapes=[
            pltpu.VMEM((_U, _D), jnp.float32),
            pltpu.VMEM((_I, _D), jnp.float32),
            pltpu.VMEM((_U, _D), jnp.float32),
            pltpu.VMEM((_I, _D), jnp.float32),
        ],
        compiler_params=pltpu.CompilerParams(
            dimension_semantics=("arbitrary", "arbitrary"),
            vmem_limit_bytes=110 * 1024 * 1024),
    )(adj, rec2, typ2, res2, user_emb_w, item_emb_w,
      recovery_emb_w, type_emb_w, resource_type_emb_w,
      user_proj_w, bu2, item_proj_w, bi2)
    return (user_out, item_out)
